# Initial kernel scaffold; baseline (speedup 1.0000x reference)
import jax
import jax.numpy as jnp
from jax.experimental import pallas as pl


def kernel(x, forgetting_factor):
    raise NotImplementedError("write your pallas kernel here")


# speedup vs baseline: 15.7069x; 15.7069x over previous
import functools
import jax
import jax.numpy as jnp
from jax import lax
from jax.experimental import pallas as pl
from jax.experimental.pallas import tpu as pltpu
from jax.experimental.pallas import tpu_sc as plsc

VOCAB = 128
N_WORDS = 8192
WORD_LEN = 32

NUM_CORES = 2
NUM_SUBCORES = 16
LANES = 16
NUM_WORKERS = NUM_CORES * NUM_SUBCORES

ROWS_PER_WORKER = N_WORDS // NUM_WORKERS
GROUPS_PER_WORKER = ROWS_PER_WORKER // LANES
XBLK = LANES * WORD_LEN
OBLK = LANES * VOCAB


def _fofe_body(x_hbm, ff_hbm, out_hbm, x_v, out_v, ff_v):
    wid = lax.axis_index("s") * NUM_CORES + lax.axis_index("c")

    pltpu.sync_copy(ff_hbm, ff_v)
    ffv = ff_v[...]
    lane = lax.iota(jnp.int32, LANES)
    row_base_idx = lane * VOCAB
    char_base_idx = lane * WORD_LEN
    zeros = jnp.zeros((LANES,), jnp.float32)

    x_word_base = wid * ROWS_PER_WORKER * WORD_LEN
    o_word_base = wid * ROWS_PER_WORKER * VOCAB

    def group_body(g, carry):
        pltpu.sync_copy(x_hbm.at[pl.ds(x_word_base + g * XBLK, XBLK)], x_v)

        for i in range(OBLK // LANES):
            out_v[pl.ds(i * LANES, LANES)] = zeros

        acc = jnp.ones((LANES,), jnp.float32)
        for j in range(WORD_LEN):
            k = WORD_LEN - 1 - j
            c = plsc.load_gather(x_v, [char_base_idx + k])
            m = c != 0
            w = jnp.where(m, acc, zeros)
            plsc.addupdate_scatter(out_v, [row_base_idx + c], w)
            acc = jnp.where(m, acc * ffv, acc)

        pltpu.sync_copy(out_v, out_hbm.at[pl.ds(o_word_base + g * OBLK, OBLK)])
        return carry

    lax.fori_loop(0, GROUPS_PER_WORKER, group_body, 0)


@jax.jit
def kernel(x, forgetting_factor):
    x_flat = x.reshape(-1).astype(jnp.int32)
    ff_vec = jnp.broadcast_to(forgetting_factor.astype(jnp.float32), (LANES,))

    mesh = plsc.VectorSubcoreMesh(
        core_axis_name="c", subcore_axis_name="s",
        num_cores=NUM_CORES, num_subcores=NUM_SUBCORES,
    )
    out_flat = pl.kernel(
        _fofe_body,
        out_type=jax.ShapeDtypeStruct((N_WORDS * VOCAB,), jnp.float32),
        mesh=mesh,
        compiler_params=pltpu.CompilerParams(needs_layout_passes=False),
        scratch_types=[
            pltpu.VMEM((XBLK,), jnp.int32),
            pltpu.VMEM((OBLK,), jnp.float32),
            pltpu.VMEM((LANES,), jnp.float32),
        ],
    )(x_flat, ff_vec)
    return out_flat.reshape(N_WORDS, VOCAB)


# speedup vs baseline: 19.9422x; 1.2696x over previous
import jax
import jax.numpy as jnp
from jax import lax
from jax.experimental import pallas as pl
from jax.experimental.pallas import tpu as pltpu
from jax.experimental.pallas import tpu_sc as plsc

VOCAB = 128
N_WORDS = 8192
WORD_LEN = 32

NUM_CORES = 2
NUM_SUBCORES = 16
LANES = 16
NUM_WORKERS = NUM_CORES * NUM_SUBCORES

ROWS_PER_WORKER = N_WORDS // NUM_WORKERS
GROUPS_PER_WORKER = ROWS_PER_WORKER // LANES
XBLK = LANES * WORD_LEN
OBLK = LANES * VOCAB
XW = ROWS_PER_WORKER * WORD_LEN
OW = ROWS_PER_WORKER * VOCAB


def _fofe_body(x_hbm, ff_hbm, out_hbm, x_v, out_v, ff_v, sem):
    wid = lax.axis_index("s") * NUM_CORES + lax.axis_index("c")

    in_cp = pltpu.async_copy(x_hbm.at[pl.ds(wid * XW, XW)], x_v, sem)
    pltpu.sync_copy(ff_hbm, ff_v)
    ffv = ff_v[...]
    lane = lax.iota(jnp.int32, LANES)
    row_base_idx = lane * VOCAB
    char_base_idx = lane * WORD_LEN
    zeros = jnp.zeros((LANES,), jnp.float32)
    ones = jnp.ones((LANES,), jnp.float32)

    def zero_body(i, carry):
        for t in range(16):
            out_v[pl.ds(i * 256 + t * LANES, LANES)] = zeros
        return carry

    lax.fori_loop(0, OW // 256, zero_body, 0)
    in_cp.wait()

    def group_body(g, carry):
        xbase = g * XBLK + char_base_idx
        obase = g * OBLK + row_base_idx
        acc = ones
        for j in range(WORD_LEN):
            k = WORD_LEN - 1 - j
            c = plsc.load_gather(x_v, [xbase + k])
            m = c != 0
            plsc.addupdate_scatter(out_v, [obase + c], acc, mask=m)
            acc = jnp.where(m, acc * ffv, acc)
        return carry

    lax.fori_loop(0, GROUPS_PER_WORKER, group_body, 0)

    pltpu.sync_copy(out_v, out_hbm.at[pl.ds(wid * OW, OW)])


@jax.jit
def kernel(x, forgetting_factor):
    x_flat = x.reshape(-1).astype(jnp.int32)
    ff_vec = jnp.broadcast_to(forgetting_factor.astype(jnp.float32), (LANES,))

    mesh = plsc.VectorSubcoreMesh(
        core_axis_name="c", subcore_axis_name="s",
        num_cores=NUM_CORES, num_subcores=NUM_SUBCORES,
    )
    out_flat = pl.kernel(
        _fofe_body,
        out_type=jax.ShapeDtypeStruct((N_WORDS * VOCAB,), jnp.float32),
        mesh=mesh,
        compiler_params=pltpu.CompilerParams(needs_layout_passes=False),
        scratch_types=[
            pltpu.VMEM((XW,), jnp.int32),
            pltpu.VMEM((OW,), jnp.float32),
            pltpu.VMEM((LANES,), jnp.float32),
            pltpu.SemaphoreType.DMA,
        ],
    )(x_flat, ff_vec)
    return out_flat.reshape(N_WORDS, VOCAB)


# speedup vs baseline: 20.1398x; 1.0099x over previous
import jax
import jax.numpy as jnp
from jax import lax
from jax.experimental import pallas as pl
from jax.experimental.pallas import tpu as pltpu
from jax.experimental.pallas import tpu_sc as plsc

VOCAB = 128
N_WORDS = 8192
WORD_LEN = 32

NUM_CORES = 2
NUM_SUBCORES = 16
LANES = 16
NUM_WORKERS = NUM_CORES * NUM_SUBCORES

ROWS_PER_WORKER = N_WORDS // NUM_WORKERS
GROUPS_PER_WORKER = ROWS_PER_WORKER // LANES


def _fofe_body(x_hbm, ff_hbm, out_hbm, x_v, out_v, ff_v, in_sem, out_sem):
    wid = lax.axis_index("s") * NUM_CORES + lax.axis_index("c")
    row0 = wid * ROWS_PER_WORKER

    in_cp = pltpu.async_copy(
        x_hbm.at[pl.ds(row0, ROWS_PER_WORKER), :], x_v, in_sem)
    pltpu.sync_copy(ff_hbm, ff_v)
    ffv = ff_v[...]
    lane = lax.iota(jnp.int32, LANES)
    zeros = jnp.zeros((LANES,), jnp.float32)
    ones = jnp.ones((LANES,), jnp.float32)
    in_cp.wait()

    def group_body(g, carry):
        r0 = g * LANES
        def zero_body(r, zcarry):
            for t in range(VOCAB // LANES):
                out_v[r0 + r, pl.ds(t * LANES, LANES)] = zeros
            return zcarry
        lax.fori_loop(0, LANES, zero_body, 0)

        rows = r0 + lane
        acc = ones
        for j in range(WORD_LEN):
            k = WORD_LEN - 1 - j
            c = plsc.load_gather(x_v, [rows, jnp.full((LANES,), k, jnp.int32)])
            m = c != 0
            plsc.addupdate_scatter(out_v, [rows, c], acc, mask=m)
            acc = jnp.where(m, acc * ffv, acc)

        pltpu.async_copy(
            out_v.at[pl.ds(r0, LANES), :],
            out_hbm.at[pl.ds(row0 + r0, LANES), :],
            out_sem)
        return carry

    lax.fori_loop(0, GROUPS_PER_WORKER, group_body, 0)

    def drain_body(g, carry):
        r0 = g * LANES
        pltpu.make_async_copy(
            out_v.at[pl.ds(r0, LANES), :],
            out_hbm.at[pl.ds(row0 + r0, LANES), :],
            out_sem).wait()
        return carry

    lax.fori_loop(0, GROUPS_PER_WORKER, drain_body, 0)


@jax.jit
def kernel(x, forgetting_factor):
    ff_vec = jnp.broadcast_to(forgetting_factor.astype(jnp.float32), (LANES,))

    mesh = plsc.VectorSubcoreMesh(
        core_axis_name="c", subcore_axis_name="s",
        num_cores=NUM_CORES, num_subcores=NUM_SUBCORES,
    )
    return pl.kernel(
        _fofe_body,
        out_type=jax.ShapeDtypeStruct((N_WORDS, VOCAB), jnp.float32),
        mesh=mesh,
        compiler_params=pltpu.CompilerParams(needs_layout_passes=False),
        scratch_types=[
            pltpu.VMEM((ROWS_PER_WORKER, WORD_LEN), jnp.int32),
            pltpu.VMEM((ROWS_PER_WORKER, VOCAB), jnp.float32),
            pltpu.VMEM((LANES,), jnp.float32),
            pltpu.SemaphoreType.DMA,
            pltpu.SemaphoreType.DMA,
        ],
    )(x, ff_vec)
